# baseline (speedup 1.0000x reference)
import functools

import jax
import jax.numpy as jnp
from jax import lax
from jax.experimental import pallas as pl
from jax.experimental.pallas import tpu as pltpu
from jax.experimental.pallas import tpu_sc as plsc

D_FEAT = 64
D_MODEL = 128
NC, NS = 2, 16
NW = NC * NS
VBLK = 1024


def _tc_project_vocab(table, W, b2d):
    v = table.shape[0]
    grid = (v + VBLK - 1) // VBLK
    mb = VBLK // 128
    tail = v - (grid - 1) * VBLK

    def body(t_hbm, w_ref, b_ref, p_ref, s_ref, tbuf, tsem):
        i = pl.program_id(0)
        slot = lax.rem(i, 2)
        nslot = lax.rem(i + 1, 2)

        def start_copy(j, s):
            @pl.when(j < grid - 1)
            def _():
                pltpu.async_copy(t_hbm.at[pl.ds(j * VBLK, VBLK)], tbuf.at[s],
                                 tsem.at[s])

            @pl.when(j == grid - 1)
            def _():
                pltpu.async_copy(t_hbm.at[pl.ds((grid - 1) * VBLK, tail)],
                                 tbuf.at[s, pl.ds(0, tail)], tsem.at[s])

        @pl.when(i == 0)
        def _():
            start_copy(0, 0)

        @pl.when(i + 1 < grid)
        def _():
            start_copy(i + 1, nslot)

        @pl.when(i < grid - 1)
        def _():
            pltpu.make_async_copy(t_hbm.at[pl.ds(0, VBLK)], tbuf.at[slot],
                                  tsem.at[slot]).wait()

        @pl.when(i == grid - 1)
        def _():
            pltpu.make_async_copy(t_hbm.at[pl.ds(0, tail)],
                                  tbuf.at[slot, pl.ds(0, tail)],
                                  tsem.at[slot]).wait()

        t = tbuf[slot]
        y = jnp.dot(t, w_ref[...], preferred_element_type=jnp.float32) + b_ref[...]
        p_ref[...] = jax.nn.gelu(y)
        s_ref[...] = jnp.sum(t.reshape(mb, 128, D_FEAT), axis=-1)

    return pl.pallas_call(
        body,
        grid=(grid,),
        in_specs=[
            pl.BlockSpec(memory_space=pl.ANY),
            pl.BlockSpec((D_FEAT, D_MODEL), lambda i: (0, 0)),
            pl.BlockSpec((1, D_MODEL), lambda i: (0, 0)),
        ],
        out_specs=[
            pl.BlockSpec((VBLK, D_MODEL), lambda i: (i, 0)),
            pl.BlockSpec((mb, 128), lambda i: (i, 0)),
        ],
        out_shape=[
            jax.ShapeDtypeStruct((v, D_MODEL), jnp.float32),
            jax.ShapeDtypeStruct((grid * mb, 128), jnp.float32),
        ],
        scratch_shapes=[
            pltpu.VMEM((2, VBLK, D_FEAT), jnp.float32),
            pltpu.SemaphoreType.DMA((2,)),
        ],
    )(table, W, b2d)


def _sc_gather(idxp, proj, seq):
    bsz, seqp = idxp.shape
    rows_w = bsz // NW
    ca, cb = 128, seq - 128
    mesh = plsc.VectorSubcoreMesh(core_axis_name="c", subcore_axis_name="s")

    @functools.partial(
        pl.kernel,
        out_type=jax.ShapeDtypeStruct((bsz * seq, D_MODEL), jnp.float32),
        mesh=mesh,
        scratch_types=[
            pltpu.VMEM((rows_w, seqp), jnp.int32),
            pltpu.VMEM((4, ca, D_MODEL), jnp.float32),
            pltpu.SemaphoreType.DMA((4,)),
            pltpu.SemaphoreType.DMA((4,)),
        ],
    )
    def k(idx_hbm, proj_hbm, out_hbm, idx_v, bufs, gsem, wsem):
        wid = lax.axis_index("s") * NC + lax.axis_index("c")
        base = wid * rows_w * seq
        pltpu.sync_copy(idx_hbm.at[pl.ds(wid * rows_w, rows_w)], idx_v)

        def ga(r, slot):
            pltpu.async_copy(proj_hbm.at[idx_v.at[r, pl.ds(0, ca)]],
                             bufs.at[slot], gsem.at[slot])

        def gb(r, slot):
            pltpu.async_copy(proj_hbm.at[idx_v.at[r, pl.ds(ca, cb)]],
                             bufs.at[slot, pl.ds(0, cb)], gsem.at[slot])

        ga(0, 0)
        gb(0, 1)

        def body(r, carry):
            pa = lax.rem(2 * r, 4)
            pb = lax.rem(2 * r + 1, 4)
            qa = lax.rem(2 * r + 2, 4)
            qb = lax.rem(2 * r + 3, 4)

            @pl.when(r + 1 < rows_w)
            def _():
                @pl.when(r >= 1)
                def _():
                    pltpu.make_async_copy(
                        bufs.at[qa], out_hbm.at[pl.ds(base, ca)],
                        wsem.at[qa]).wait()

                ga(r + 1, qa)

                @pl.when(r >= 1)
                def _():
                    pltpu.make_async_copy(
                        bufs.at[qb, pl.ds(0, cb)], out_hbm.at[pl.ds(base, cb)],
                        wsem.at[qb]).wait()

                gb(r + 1, qb)

            pltpu.make_async_copy(proj_hbm.at[idx_v.at[r, pl.ds(0, ca)]],
                                  bufs.at[pa], gsem.at[pa]).wait()
            pltpu.async_copy(bufs.at[pa],
                             out_hbm.at[pl.ds(base + r * seq, ca)],
                             wsem.at[pa])
            pltpu.make_async_copy(proj_hbm.at[idx_v.at[r, pl.ds(ca, cb)]],
                                  bufs.at[pb, pl.ds(0, cb)], gsem.at[pb]).wait()
            pltpu.async_copy(bufs.at[pb, pl.ds(0, cb)],
                             out_hbm.at[pl.ds(base + r * seq + ca, cb)],
                             wsem.at[pb])
            return carry

        lax.fori_loop(0, rows_w, body, 0)
        for p in (0, 2):
            pltpu.make_async_copy(bufs.at[p], out_hbm.at[pl.ds(base, ca)],
                                  wsem.at[p]).wait()
        for p in (1, 3):
            pltpu.make_async_copy(bufs.at[p, pl.ds(0, cb)],
                                  out_hbm.at[pl.ds(base, cb)],
                                  wsem.at[p]).wait()

    return k(idxp, proj)


def _sc_mask(idxp, rs_flat, seq):
    bsz, seqp = idxp.shape
    rows_w = bsz // NW
    flat_w = rows_w * seq
    gpr = (seq + 15) // 16
    last_off = seq - 16
    mesh = plsc.VectorSubcoreMesh(core_axis_name="c", subcore_axis_name="s")

    @functools.partial(
        pl.kernel,
        out_type=jax.ShapeDtypeStruct((bsz * seq,), jnp.float32),
        mesh=mesh,
        scratch_types=[
            pltpu.VMEM(rs_flat.shape, jnp.float32),
            pltpu.VMEM((rows_w, seqp), jnp.int32),
            pltpu.VMEM((flat_w,), jnp.float32),
        ],
        compiler_params=pltpu.CompilerParams(needs_layout_passes=False),
    )
    def k(idx_hbm, rs_hbm, out_hbm, rs_v, idx_v, m_v):
        wid = lax.axis_index("s") * NC + lax.axis_index("c")
        pltpu.sync_copy(rs_hbm, rs_v)
        pltpu.sync_copy(idx_hbm.at[pl.ds(wid * rows_w, rows_w)], idx_v)

        def body(g, carry):
            r = g // gpr
            c = g - r * gpr
            off = jnp.minimum(c * 16, last_off)
            vidx = idx_v[r, pl.ds(off, 16)]
            vals = plsc.load_gather(rs_v, [vidx])
            m_v[pl.ds(r * seq + off, 16)] = jnp.where(
                vals == 0.0, 1.0, 0.0).astype(jnp.float32)
            return carry

        lax.fori_loop(0, rows_w * gpr, body, 0)
        pltpu.sync_copy(m_v, out_hbm.at[pl.ds(wid * flat_w, flat_w)])

    return k(idxp, rs_flat)


def kernel(indices, table, W, b):
    bsz, seq = indices.shape
    seqp = ((seq + 127) // 128) * 128
    idxp = jnp.pad(indices.astype(jnp.int32), ((0, 0), (0, seqp - seq)))
    proj, rowsum = _tc_project_vocab(table, W, b.reshape(1, D_MODEL))
    out_flat = _sc_gather(idxp, proj, seq)
    mask_flat = _sc_mask(idxp, rowsum.reshape(-1), seq)
    out = out_flat.reshape(bsz, seq, D_MODEL)
    mask = mask_flat.reshape(bsz, seq).astype(bool)[:, None, None, :]
    return out, mask


# speedup vs baseline: 1.1193x; 1.1193x over previous
import functools

import jax
import jax.numpy as jnp
from jax import lax
from jax.experimental import pallas as pl
from jax.experimental.pallas import tpu as pltpu
from jax.experimental.pallas import tpu_sc as plsc

D_FEAT = 64
D_MODEL = 128
NC, NS = 2, 16
NW = NC * NS
VBLK = 2048


def _tc_project_vocab(table, W, b2d):
    v = table.shape[0]
    grid = (v + VBLK - 1) // VBLK
    mb = VBLK // 128
    tail = v - (grid - 1) * VBLK

    def body(t_hbm, w_ref, b_ref, p_ref, s_ref, tbuf, tsem):
        i = pl.program_id(0)
        slot = lax.rem(i, 2)
        nslot = lax.rem(i + 1, 2)

        def start_copy(j, s):
            @pl.when(j < grid - 1)
            def _():
                pltpu.async_copy(t_hbm.at[pl.ds(j * VBLK, VBLK)], tbuf.at[s],
                                 tsem.at[s])

            @pl.when(j == grid - 1)
            def _():
                pltpu.async_copy(t_hbm.at[pl.ds((grid - 1) * VBLK, tail)],
                                 tbuf.at[s, pl.ds(0, tail)], tsem.at[s])

        @pl.when(i == 0)
        def _():
            start_copy(0, 0)

        @pl.when(i + 1 < grid)
        def _():
            start_copy(i + 1, nslot)

        @pl.when(i < grid - 1)
        def _():
            pltpu.make_async_copy(t_hbm.at[pl.ds(0, VBLK)], tbuf.at[slot],
                                  tsem.at[slot]).wait()

        @pl.when(i == grid - 1)
        def _():
            pltpu.make_async_copy(t_hbm.at[pl.ds(0, tail)],
                                  tbuf.at[slot, pl.ds(0, tail)],
                                  tsem.at[slot]).wait()

        t = tbuf[slot]
        y = jnp.dot(t, w_ref[...], preferred_element_type=jnp.float32) + b_ref[...]
        p_ref[...] = jax.nn.gelu(y)
        s_ref[...] = jnp.sum(t.reshape(mb, 128, D_FEAT), axis=-1)

    return pl.pallas_call(
        body,
        grid=(grid,),
        in_specs=[
            pl.BlockSpec(memory_space=pl.ANY),
            pl.BlockSpec((D_FEAT, D_MODEL), lambda i: (0, 0)),
            pl.BlockSpec((1, D_MODEL), lambda i: (0, 0)),
        ],
        out_specs=[
            pl.BlockSpec((VBLK, D_MODEL), lambda i: (i, 0)),
            pl.BlockSpec((mb, 128), lambda i: (i, 0)),
        ],
        out_shape=[
            jax.ShapeDtypeStruct((v, D_MODEL), jnp.float32),
            jax.ShapeDtypeStruct((grid * mb, 128), jnp.float32),
        ],
        scratch_shapes=[
            pltpu.VMEM((2, VBLK, D_FEAT), jnp.float32),
            pltpu.SemaphoreType.DMA((2,)),
        ],
    )(table, W, b2d)


def _sc_gather(idxp, proj, seq):
    bsz, seqp = idxp.shape
    rows_w = bsz // NW
    ca, cb = 128, seq - 128
    mesh = plsc.VectorSubcoreMesh(core_axis_name="c", subcore_axis_name="s")

    @functools.partial(
        pl.kernel,
        out_type=jax.ShapeDtypeStruct((bsz * seq, D_MODEL), jnp.float32),
        mesh=mesh,
        scratch_types=[
            pltpu.VMEM((rows_w, seqp), jnp.int32),
            pltpu.VMEM((4, ca, D_MODEL), jnp.float32),
            pltpu.SemaphoreType.DMA((4,)),
            pltpu.SemaphoreType.DMA((4,)),
        ],
    )
    def k(idx_hbm, proj_hbm, out_hbm, idx_v, bufs, gsem, wsem):
        wid = lax.axis_index("s") * NC + lax.axis_index("c")
        base = wid * rows_w * seq
        pltpu.sync_copy(idx_hbm.at[pl.ds(wid * rows_w, rows_w)], idx_v)

        def ga(r, slot):
            pltpu.async_copy(proj_hbm.at[idx_v.at[r, pl.ds(0, ca)]],
                             bufs.at[slot], gsem.at[slot])

        def gb(r, slot):
            pltpu.async_copy(proj_hbm.at[idx_v.at[r, pl.ds(ca, cb)]],
                             bufs.at[slot, pl.ds(0, cb)], gsem.at[slot])

        ga(0, 0)
        gb(0, 1)

        def body(r, carry):
            pa = lax.rem(2 * r, 4)
            pb = lax.rem(2 * r + 1, 4)
            qa = lax.rem(2 * r + 2, 4)
            qb = lax.rem(2 * r + 3, 4)

            @pl.when(r + 1 < rows_w)
            def _():
                @pl.when(r >= 1)
                def _():
                    pltpu.make_async_copy(
                        bufs.at[qa], out_hbm.at[pl.ds(base, ca)],
                        wsem.at[qa]).wait()

                ga(r + 1, qa)

                @pl.when(r >= 1)
                def _():
                    pltpu.make_async_copy(
                        bufs.at[qb, pl.ds(0, cb)], out_hbm.at[pl.ds(base, cb)],
                        wsem.at[qb]).wait()

                gb(r + 1, qb)

            pltpu.make_async_copy(proj_hbm.at[idx_v.at[r, pl.ds(0, ca)]],
                                  bufs.at[pa], gsem.at[pa]).wait()
            pltpu.async_copy(bufs.at[pa],
                             out_hbm.at[pl.ds(base + r * seq, ca)],
                             wsem.at[pa])
            pltpu.make_async_copy(proj_hbm.at[idx_v.at[r, pl.ds(ca, cb)]],
                                  bufs.at[pb, pl.ds(0, cb)], gsem.at[pb]).wait()
            pltpu.async_copy(bufs.at[pb, pl.ds(0, cb)],
                             out_hbm.at[pl.ds(base + r * seq + ca, cb)],
                             wsem.at[pb])
            return carry

        lax.fori_loop(0, rows_w, body, 0)
        for p in (0, 2):
            pltpu.make_async_copy(bufs.at[p], out_hbm.at[pl.ds(base, ca)],
                                  wsem.at[p]).wait()
        for p in (1, 3):
            pltpu.make_async_copy(bufs.at[p, pl.ds(0, cb)],
                                  out_hbm.at[pl.ds(base, cb)],
                                  wsem.at[p]).wait()

    return k(idxp, proj)


def _sc_mask(idxp, rs_flat, seq):
    bsz, seqp = idxp.shape
    rows_w = bsz // NW
    flat_w = rows_w * seq
    gpr = (seq + 15) // 16
    last_off = seq - 16
    mesh = plsc.VectorSubcoreMesh(core_axis_name="c", subcore_axis_name="s")

    @functools.partial(
        pl.kernel,
        out_type=jax.ShapeDtypeStruct((bsz * seq,), jnp.float32),
        mesh=mesh,
        scratch_types=[
            pltpu.VMEM(rs_flat.shape, jnp.float32),
            pltpu.VMEM((rows_w, seqp), jnp.int32),
            pltpu.VMEM((flat_w,), jnp.float32),
        ],
        compiler_params=pltpu.CompilerParams(needs_layout_passes=False),
    )
    def k(idx_hbm, rs_hbm, out_hbm, rs_v, idx_v, m_v):
        wid = lax.axis_index("s") * NC + lax.axis_index("c")
        pltpu.sync_copy(rs_hbm, rs_v)
        pltpu.sync_copy(idx_hbm.at[pl.ds(wid * rows_w, rows_w)], idx_v)

        def body(g, carry):
            r = g // gpr
            c = g - r * gpr
            off = jnp.minimum(c * 16, last_off)
            vidx = idx_v[r, pl.ds(off, 16)]
            vals = plsc.load_gather(rs_v, [vidx])
            m_v[pl.ds(r * seq + off, 16)] = jnp.where(
                vals == 0.0, 1.0, 0.0).astype(jnp.float32)
            return carry

        lax.fori_loop(0, rows_w * gpr, body, 0)
        pltpu.sync_copy(m_v, out_hbm.at[pl.ds(wid * flat_w, flat_w)])

    return k(idxp, rs_flat)


def kernel(indices, table, W, b):
    bsz, seq = indices.shape
    seqp = ((seq + 127) // 128) * 128
    idxp = jnp.pad(indices.astype(jnp.int32), ((0, 0), (0, seqp - seq)))
    proj, rowsum = _tc_project_vocab(table, W, b.reshape(1, D_MODEL))
    out_flat = _sc_gather(idxp, proj, seq)
    mask_flat = _sc_mask(idxp, rowsum.reshape(-1), seq)
    out = out_flat.reshape(bsz, seq, D_MODEL)
    mask = mask_flat.reshape(bsz, seq).astype(bool)[:, None, None, :]
    return out, mask


# speedup vs baseline: 1.1208x; 1.0014x over previous
import functools

import jax
import jax.numpy as jnp
from jax import lax
from jax.experimental import pallas as pl
from jax.experimental.pallas import tpu as pltpu
from jax.experimental.pallas import tpu_sc as plsc

D_FEAT = 64
D_MODEL = 128
NC, NS = 2, 16
NW = NC * NS
VBLK = 2048


def _tc_project_vocab(table, W, b2d):
    v = table.shape[0]
    grid = (v + VBLK - 1) // VBLK
    mb = VBLK // 128
    tail = v - (grid - 1) * VBLK

    def body(t_ref, w_ref, b_ref, p_ref, s_ref):
        t = t_ref[...]
        y = jnp.dot(t, w_ref[...], preferred_element_type=jnp.float32) + b_ref[...]
        p_ref[...] = jax.nn.gelu(y)
        s_ref[...] = jnp.sum(t.reshape(mb, 128, D_FEAT), axis=-1)

    return pl.pallas_call(
        body,
        grid=(grid,),
        in_specs=[
            pl.BlockSpec((VBLK, D_FEAT), lambda i: (i, 0)),
            pl.BlockSpec((D_FEAT, D_MODEL), lambda i: (0, 0)),
            pl.BlockSpec((1, D_MODEL), lambda i: (0, 0)),
        ],
        out_specs=[
            pl.BlockSpec((VBLK, D_MODEL), lambda i: (i, 0)),
            pl.BlockSpec((mb, 128), lambda i: (i, 0)),
        ],
        out_shape=[
            jax.ShapeDtypeStruct((v, D_MODEL), jnp.float32),
            jax.ShapeDtypeStruct((grid * mb, 128), jnp.float32),
        ],
    )(table, W, b2d)


def _sc_gather(idxp, proj, seq):
    bsz, seqp = idxp.shape
    rows_w = bsz // NW
    ca, cb = 128, seq - 128
    mesh = plsc.VectorSubcoreMesh(core_axis_name="c", subcore_axis_name="s")

    @functools.partial(
        pl.kernel,
        out_type=jax.ShapeDtypeStruct((bsz * seq, D_MODEL), jnp.float32),
        mesh=mesh,
        scratch_types=[
            pltpu.VMEM((rows_w, seqp), jnp.int32),
            pltpu.VMEM((4, ca, D_MODEL), jnp.float32),
            pltpu.SemaphoreType.DMA((4,)),
            pltpu.SemaphoreType.DMA((4,)),
        ],
    )
    def k(idx_hbm, proj_hbm, out_hbm, idx_v, bufs, gsem, wsem):
        wid = lax.axis_index("s") * NC + lax.axis_index("c")
        base = wid * rows_w * seq
        pltpu.sync_copy(idx_hbm.at[pl.ds(wid * rows_w, rows_w)], idx_v)

        def ga(r, slot):
            pltpu.async_copy(proj_hbm.at[idx_v.at[r, pl.ds(0, ca)]],
                             bufs.at[slot], gsem.at[slot])

        def gb(r, slot):
            pltpu.async_copy(proj_hbm.at[idx_v.at[r, pl.ds(ca, cb)]],
                             bufs.at[slot, pl.ds(0, cb)], gsem.at[slot])

        ga(0, 0)
        gb(0, 1)

        def body(r, carry):
            pa = lax.rem(2 * r, 4)
            pb = lax.rem(2 * r + 1, 4)
            qa = lax.rem(2 * r + 2, 4)
            qb = lax.rem(2 * r + 3, 4)

            @pl.when(r + 1 < rows_w)
            def _():
                @pl.when(r >= 1)
                def _():
                    pltpu.make_async_copy(
                        bufs.at[qa], out_hbm.at[pl.ds(base, ca)],
                        wsem.at[qa]).wait()

                ga(r + 1, qa)

                @pl.when(r >= 1)
                def _():
                    pltpu.make_async_copy(
                        bufs.at[qb, pl.ds(0, cb)], out_hbm.at[pl.ds(base, cb)],
                        wsem.at[qb]).wait()

                gb(r + 1, qb)

            pltpu.make_async_copy(proj_hbm.at[idx_v.at[r, pl.ds(0, ca)]],
                                  bufs.at[pa], gsem.at[pa]).wait()
            pltpu.async_copy(bufs.at[pa],
                             out_hbm.at[pl.ds(base + r * seq, ca)],
                             wsem.at[pa])
            pltpu.make_async_copy(proj_hbm.at[idx_v.at[r, pl.ds(ca, cb)]],
                                  bufs.at[pb, pl.ds(0, cb)], gsem.at[pb]).wait()
            pltpu.async_copy(bufs.at[pb, pl.ds(0, cb)],
                             out_hbm.at[pl.ds(base + r * seq + ca, cb)],
                             wsem.at[pb])
            return carry

        lax.fori_loop(0, rows_w, body, 0)
        for p in (0, 2):
            pltpu.make_async_copy(bufs.at[p], out_hbm.at[pl.ds(base, ca)],
                                  wsem.at[p]).wait()
        for p in (1, 3):
            pltpu.make_async_copy(bufs.at[p, pl.ds(0, cb)],
                                  out_hbm.at[pl.ds(base, cb)],
                                  wsem.at[p]).wait()

    return k(idxp, proj)


def _sc_mask(idxp, rs_flat, seq):
    bsz, seqp = idxp.shape
    rows_w = bsz // NW
    flat_w = rows_w * seq
    gpr = (seq + 15) // 16
    last_off = seq - 16
    mesh = plsc.VectorSubcoreMesh(core_axis_name="c", subcore_axis_name="s")

    @functools.partial(
        pl.kernel,
        out_type=jax.ShapeDtypeStruct((bsz * seq,), jnp.float32),
        mesh=mesh,
        scratch_types=[
            pltpu.VMEM(rs_flat.shape, jnp.float32),
            pltpu.VMEM((rows_w, seqp), jnp.int32),
            pltpu.VMEM((flat_w,), jnp.float32),
        ],
        compiler_params=pltpu.CompilerParams(needs_layout_passes=False),
    )
    def k(idx_hbm, rs_hbm, out_hbm, rs_v, idx_v, m_v):
        wid = lax.axis_index("s") * NC + lax.axis_index("c")
        pltpu.sync_copy(rs_hbm, rs_v)
        pltpu.sync_copy(idx_hbm.at[pl.ds(wid * rows_w, rows_w)], idx_v)

        def body(g, carry):
            r = g // gpr
            c = g - r * gpr
            off = jnp.minimum(c * 16, last_off)
            vidx = idx_v[r, pl.ds(off, 16)]
            vals = plsc.load_gather(rs_v, [vidx])
            m_v[pl.ds(r * seq + off, 16)] = jnp.where(
                vals == 0.0, 1.0, 0.0).astype(jnp.float32)
            return carry

        lax.fori_loop(0, rows_w * gpr, body, 0)
        pltpu.sync_copy(m_v, out_hbm.at[pl.ds(wid * flat_w, flat_w)])

    return k(idxp, rs_flat)


def kernel(indices, table, W, b):
    bsz, seq = indices.shape
    seqp = ((seq + 127) // 128) * 128
    idxp = jnp.pad(indices.astype(jnp.int32), ((0, 0), (0, seqp - seq)))
    proj, rowsum = _tc_project_vocab(table, W, b.reshape(1, D_MODEL))
    out_flat = _sc_gather(idxp, proj, seq)
    mask_flat = _sc_mask(idxp, rowsum.reshape(-1), seq)
    out = out_flat.reshape(bsz, seq, D_MODEL)
    mask = mask_flat.reshape(bsz, seq).astype(bool)[:, None, None, :]
    return out, mask
